# Initial kernel scaffold; baseline (speedup 1.0000x reference)
import jax
import jax.numpy as jnp
from jax.experimental import pallas as pl


def kernel(x, edge_index, edge_attr, batch, W_ae, b_ae, W_e1, b_e1, W_e2, b_e2, W_root, b_conv, W_ih, W_hh, b_ih, b_hh, Wl_ih, Wl_hh, bl_ih, bl_hh):
    raise NotImplementedError("write your pallas kernel here")


# speedup vs baseline: 1.3615x; 1.3615x over previous
import functools

import jax
import jax.numpy as jnp
from jax import lax
from jax.experimental import pallas as pl
from jax.experimental.pallas import tpu as pltpu
from jax.experimental.pallas import tpu_sc as plsc

F32 = jnp.float32


def _dot(a, b):
    return jnp.dot(a, b, preferred_element_type=F32)


def kernel(x, edge_index, edge_attr, batch, W_ae, b_ae, W_e1, b_e1, W_e2,
           b_e2, W_root, b_conv, W_ih, W_hh, b_ih, b_hh, Wl_ih, Wl_hh,
           bl_ih, bl_hh):
    N, NODE_DIM = x.shape
    E, BOND = edge_attr.shape
    D = W_ae.shape[1]
    HID = W_e1.shape[1]
    B = 256
    DEPTH = 3
    STEPS = 3

    NC, NS = 2, 16
    NW = NC * NS
    CPT = 10
    CH = 1024
    EPW = CPT * CH
    EP = NW * EPW

    mesh = plsc.VectorSubcoreMesh(core_axis_name="c", subcore_axis_name="s")

    @functools.partial(
        pl.kernel,
        out_type=jax.ShapeDtypeStruct((EP, D), F32),
        mesh=mesh,
        scratch_types=[
            pltpu.VMEM((8, 128), jnp.int32),
            pltpu.VMEM((128, 128), F32),
            pltpu.VMEM((512, D), F32),
            pltpu.SemaphoreType.DMA,
        ],
    )
    def sc_gather(tab_hbm, idx_hbm, out_hbm, idx_v, rows_v, cmp_v, sem):
        cid = lax.axis_index("c")
        sid = lax.axis_index("s")
        wid = sid * NC + cid

        def chunk(c, carry):
            pltpu.sync_copy(idx_hbm.at[pl.ds((wid * CPT + c) * 8, 8)], idx_v)
            for sb in range(2):
                for g in range(4):
                    pltpu.async_copy(tab_hbm.at[idx_v.at[sb * 4 + g]],
                                     rows_v, sem).wait()

                    def cpy(r, cc):
                        cmp_v[g * 128 + r] = rows_v[r, pl.ds(0, D)]
                        return cc
                    lax.fori_loop(0, 128, cpy, 0)
                pltpu.sync_copy(
                    cmp_v,
                    out_hbm.at[pl.ds(wid * EPW + c * CH + sb * 512, 512)])
            return carry

        lax.fori_loop(0, CPT, chunk, 0)

    def _full(a):
        return pl.BlockSpec(a.shape, lambda i: (0,) * a.ndim)

    TN = 2000
    GN = N // TN
    TEB = 4096
    GE = EP // TEB

    def init_body(xr, w, b, o, ow):
        sv = jnp.maximum(_dot(xr[...], w[...]) + b[...], 0.0)
        o[...] = sv
        ow[...] = jnp.pad(sv, ((0, 0), (0, 128 - D)))

    b_ae2 = b_ae.reshape(1, D)
    init_call = pl.pallas_call(
        init_body,
        grid=(GN,),
        in_specs=[pl.BlockSpec((TN, NODE_DIM), lambda i: (i, 0)),
                  _full(W_ae), _full(b_ae2)],
        out_specs=(pl.BlockSpec((TN, D), lambda i: (i, 0)),
                   pl.BlockSpec((TN, 128), lambda i: (i, 0))),
        out_shape=(jax.ShapeDtypeStruct((N, D), F32),
                   jax.ShapeDtypeStruct((N, 128), F32)),
    )

    def msg_body(ea, xs, w1, b1, w2, b2, rm, sm, o):
        t = jnp.maximum(_dot(ea[...], w1[...]) + b1[...], 0.0)
        ew = _dot(t, w2[...]) + b2[...]
        xst = _dot(xs[...], rm[...])
        o[...] = _dot(xst * ew, sm[...])

    b_e1r = b_e1.reshape(1, HID)
    b_e2r = b_e2.reshape(1, D * D)
    Rm = jnp.repeat(jnp.eye(D, dtype=F32), D, axis=1)
    Sm = jnp.tile(jnp.eye(D, dtype=F32), (D, 1))
    msg_call = pl.pallas_call(
        msg_body,
        grid=(GE,),
        in_specs=[pl.BlockSpec((TEB, BOND), lambda i: (i, 0)),
                  pl.BlockSpec((TEB, D), lambda i: (i, 0)),
                  _full(W_e1), _full(b_e1r), _full(W_e2), _full(b_e2r),
                  _full(Rm), _full(Sm)],
        out_specs=pl.BlockSpec((TEB, D), lambda i: (i, 0)),
        out_shape=jax.ShapeDtypeStruct((EP, D), F32),
    )

    def upd_body(agg, cnt, s, wroot, bconv, wir, wiz, win, whr, whz, whn,
                 bir, biz, binn, bhr, bhz, bhn, o, ow):
        ag = agg[...] / jnp.maximum(cnt[...], 1.0)
        sv = s[...]
        m2 = jnp.maximum(ag + _dot(sv, wroot[...]) + bconv[...], 0.0)
        r = jax.nn.sigmoid(_dot(m2, wir[...]) + bir[...]
                           + _dot(sv, whr[...]) + bhr[...])
        z = jax.nn.sigmoid(_dot(m2, wiz[...]) + biz[...]
                           + _dot(sv, whz[...]) + bhz[...])
        nn = jnp.tanh(_dot(m2, win[...]) + binn[...]
                      + r * (_dot(sv, whn[...]) + bhn[...]))
        sv2 = (1.0 - z) * nn + z * sv
        o[...] = sv2
        ow[...] = jnp.pad(sv2, ((0, 0), (0, 128 - D)))

    Wir, Wiz, Win = jnp.split(W_ih, 3, axis=1)
    Whr, Whz, Whn = jnp.split(W_hh, 3, axis=1)
    bir, biz, binn = [v.reshape(1, D) for v in jnp.split(b_ih, 3)]
    bhr, bhz, bhn = [v.reshape(1, D) for v in jnp.split(b_hh, 3)]
    b_convr = b_conv.reshape(1, D)
    upd_call = pl.pallas_call(
        upd_body,
        grid=(GN,),
        in_specs=[pl.BlockSpec((TN, D), lambda i: (i, 0)),
                  pl.BlockSpec((TN, D), lambda i: (i, 0)),
                  pl.BlockSpec((TN, D), lambda i: (i, 0)),
                  _full(W_root), _full(b_convr),
                  _full(Wir), _full(Wiz), _full(Win),
                  _full(Whr), _full(Whz), _full(Whn),
                  _full(bir), _full(biz), _full(binn),
                  _full(bhr), _full(bhz), _full(bhn)],
        out_specs=(pl.BlockSpec((TN, D), lambda i: (i, 0)),
                   pl.BlockSpec((TN, 128), lambda i: (i, 0))),
        out_shape=(jax.ShapeDtypeStruct((N, D), F32),
                   jax.ShapeDtypeStruct((N, 128), F32)),
    )

    def s2s_body(sref, bref, wq, wr_, wh, bl, qo, ro):
        out = sref[...]
        A = (bref[...] == lax.broadcasted_iota(jnp.int32, (N, B), 1)
             ).astype(F32)
        q = jnp.zeros((B, D), F32)
        rv = jnp.zeros((B, D), F32)
        hl = jnp.zeros((B, D), F32)
        cl = jnp.zeros((B, D), F32)
        for _ in range(STEPS):
            gates = (_dot(q, wq[...]) + _dot(rv, wr_[...])
                     + _dot(hl, wh[...]) + bl[...])
            ii = gates[:, 0:D]
            ff = gates[:, D:2 * D]
            gg = gates[:, 2 * D:3 * D]
            oo = gates[:, 3 * D:4 * D]
            cl = jax.nn.sigmoid(ff) * cl + jax.nn.sigmoid(ii) * jnp.tanh(gg)
            hl = jax.nn.sigmoid(oo) * jnp.tanh(cl)
            q = hl
            qb = _dot(A, q)
            e = jnp.sum(out * qb, axis=1, keepdims=True)
            M = jnp.where(A > 0, e, -1e30)
            emax = jnp.max(M, axis=0, keepdims=True)
            EE = jnp.where(A > 0, jnp.exp(M - emax), 0.0)
            denom = jnp.sum(EE, axis=0, keepdims=True)
            Am = EE / (denom + 1e-16)
            rv = lax.dot_general(Am, out, (((0,), (0,)), ((), ())),
                                 preferred_element_type=F32)
        qo[...] = q
        ro[...] = rv

    batch2 = batch.reshape(N, 1)
    s2s_call = pl.pallas_call(
        s2s_body,
        out_shape=(jax.ShapeDtypeStruct((B, D), F32),
                   jax.ShapeDtypeStruct((B, D), F32)),
    )
    Wq = Wl_ih[:D]
    Wr_ = Wl_ih[D:]
    bl2 = (bl_ih + bl_hh).reshape(1, 4 * D)

    src2 = jnp.concatenate(
        [edge_index[0], jnp.zeros((EP - E,), jnp.int32)]).reshape(
            NW * CPT * 8, 128)
    dst = edge_index[1]
    ea_p = jnp.concatenate(
        [edge_attr, jnp.zeros((EP - E, BOND), F32)], axis=0)

    cnt = jax.ops.segment_sum(jnp.ones((E,), F32), dst, num_segments=N)
    cntb = jnp.broadcast_to(cnt[:, None], (N, D))

    s, sw = init_call(x, W_ae, b_ae2)
    for _ in range(DEPTH):
        xs = sc_gather(sw, src2)
        m = msg_call(ea_p, xs, W_e1, b_e1r, W_e2, b_e2r, Rm, Sm)
        agg = jax.ops.segment_sum(m[:E], dst, num_segments=N)
        s, sw = upd_call(agg, cntb, s, W_root, b_convr, Wir, Wiz, Win,
                         Whr, Whz, Whn, bir, biz, binn, bhr, bhz, bhn)
    q, rv = s2s_call(s, batch2, Wq, Wr_, Wl_hh, bl2)
    return jnp.concatenate([q, rv], axis=-1)
